# baseline (speedup 1.0000x reference)
import functools

import jax
import jax.numpy as jnp
from jax import lax
from jax.experimental import pallas as pl
from jax.experimental.pallas import tpu as pltpu

B = 4096
T = 200
D = 128
V = 60

BB = 64
HALF = B // 2


def _tc_body(t_ref, table_ref, out_ref):
    idx = t_ref[0, 0, :]
    onehot = (idx[:, None] == jax.lax.broadcasted_iota(jnp.int32, (BB, V), 1)
              ).astype(jnp.float32)
    emb = jnp.dot(onehot, table_ref[...], preferred_element_type=jnp.float32)
    out_ref[...] = jnp.broadcast_to(emb[:, None, :], (BB, T, D))


def _half(t_half, table):
    grid = HALF // BB
    t3 = t_half.reshape(grid, 1, BB)
    return pl.pallas_call(
        _tc_body,
        grid=(grid,),
        in_specs=[
            pl.BlockSpec((1, 1, BB), lambda i: (i, 0, 0)),
            pl.BlockSpec((V, D), lambda i: (0, 0)),
        ],
        out_specs=pl.BlockSpec((BB, T, D), lambda i: (i, 0, 0)),
        out_shape=jax.ShapeDtypeStruct((HALF, T, D), jnp.float32),
    )(t3, table)


@jax.jit
def _run(t, table):
    out0 = _half(t[:HALF], table)
    out1 = _half(t[HALF:], table)
    return jnp.concatenate([out0, out1], axis=0)


def kernel(t, n_tokens, table):
    del n_tokens
    return _run(t, table)


# speedup vs baseline: 2.4815x; 2.4815x over previous
import functools

import jax
import jax.numpy as jnp
from jax import lax
from jax.experimental import pallas as pl
from jax.experimental.pallas import tpu as pltpu
from jax.experimental.pallas import tpu_sc as plsc

B = 4096
T = 200
D = 128

_INFO = plsc.get_sparse_core_info()
NC = _INFO.num_cores
NS = _INFO.num_subcores
NW = NC * NS
BPW = B // NW

_MESH = plsc.VectorSubcoreMesh(core_axis_name="c", subcore_axis_name="s")


@functools.partial(
    pl.kernel,
    mesh=_MESH,
    out_type=jax.ShapeDtypeStruct((B, T, D), jnp.float32),
    scratch_types=[
        pltpu.VMEM((BPW,), jnp.int32),
        pltpu.VMEM((BPW, D), jnp.float32),
        pltpu.VMEM((2, T, D), jnp.float32),
        pltpu.SemaphoreType.DMA,
        pltpu.SemaphoreType.DMA,
        pltpu.SemaphoreType.DMA,
    ],
)
def _sc_expand(t_hbm, table_hbm, out_hbm, idx_v, rows_v, exp_v, gsem, sem0, sem1):
    wid = lax.axis_index("s") * NC + lax.axis_index("c")
    base = wid * BPW
    pltpu.sync_copy(t_hbm.at[pl.ds(base, BPW)], idx_v)
    pltpu.async_copy(table_hbm.at[idx_v], rows_v, gsem).wait()

    sems = (sem0, sem1)

    def _build(buf, b):
        vecs = [rows_v[b, pl.ds(k * 16, 16)] for k in range(D // 16)]

        def fill(j, _):
            for k in range(D // 16):
                exp_v[buf, j, pl.ds(k * 16, 16)] = vecs[k]
            return 0

        lax.fori_loop(0, T, fill, 0)

    def _start(buf, b):
        pltpu.async_copy(exp_v.at[buf], out_hbm.at[base + b], sems[buf])

    def _wait(buf):
        pltpu.make_async_copy(exp_v.at[buf], out_hbm.at[base], sems[buf]).wait()

    def body(pair, _):
        for par in range(2):
            b = pair * 2 + par

            @pl.when(pair > 0)
            def _():
                _wait(par)

            _build(par, b)
            _start(par, b)
        return 0

    lax.fori_loop(0, BPW // 2, body, 0)
    _wait(0)
    _wait(1)


def kernel(t, n_tokens, table):
    del n_tokens
    return _sc_expand(t, table)
